# baseline (speedup 1.0000x reference)
import functools

import jax
import jax.numpy as jnp
from jax import lax
from jax.experimental import pallas as pl
from jax.experimental.pallas import tpu as pltpu
from jax.experimental.pallas import tpu_sc as plsc

_B = 16384
_D = 64
_NE = 1000000
_NR = 1000

_info = plsc.get_sparse_core_info()
_NC = _info.num_cores
_NS = _info.num_subcores
_NW = _NC * _NS
_BPW = _B // _NW
_CH = 64
_NCHUNK = _BPW // _CH

_mesh = plsc.VectorSubcoreMesh(core_axis_name="c", subcore_axis_name="s")


@functools.partial(
    pl.kernel,
    mesh=_mesh,
    out_type=[
        jax.ShapeDtypeStruct((_B, _D), jnp.float32),
        jax.ShapeDtypeStruct((_B, _D), jnp.float32),
        jax.ShapeDtypeStruct((_B, _D), jnp.float32),
    ],
    scratch_types=[
        pltpu.VMEM((_BPW,), jnp.int32),
        pltpu.VMEM((_BPW,), jnp.int32),
        pltpu.VMEM((_BPW,), jnp.int32),
        pltpu.VMEM((_CH, _D), jnp.float32),
        pltpu.VMEM((_CH, _D), jnp.float32),
        pltpu.VMEM((_CH, _D), jnp.float32),
        pltpu.VMEM((_NR * _D,), jnp.float32),
        pltpu.SemaphoreType.DMA,
        pltpu.SemaphoreType.DMA,
    ],
)
def _gather3(ent_hbm, rel_hbm, h_hbm, r_hbm, t_hbm,
             h_out, r_out, t_out,
             sh_s, st_s, sr_s,
             oh, ot, orel, rel_v,
             sem_h, sem_t):
    wid = lax.axis_index("s") * _NC + lax.axis_index("c")
    base = wid * _BPW

    pltpu.sync_copy(h_hbm.at[pl.ds(base, _BPW)], sh_s)
    pltpu.sync_copy(t_hbm.at[pl.ds(base, _BPW)], st_s)
    pltpu.sync_copy(r_hbm.at[pl.ds(base, _BPW)], sr_s)
    pltpu.sync_copy(rel_hbm, rel_v)

    @pl.loop(0, _NCHUNK)
    def _chunk(ci):
        cbase = ci * _CH

        @pl.loop(0, _CH // 16)
        def _fire(g):
            s = pl.ds(cbase + g * 16, 16)
            hv = sh_s[s]
            tv = st_s[s]
            rv = sr_s[s] * _D
            for l in range(16):
                j = g * 16 + l
                pltpu.async_copy(ent_hbm.at[hv[l]], oh.at[j], sem_h)
                pltpu.async_copy(ent_hbm.at[tv[l]], ot.at[j], sem_t)
                rbase = rv[l]
                for b in range(_D // 16):
                    orel[j, pl.ds(b * 16, 16)] = rel_v[pl.ds(rbase + b * 16, 16)]

        pltpu.make_async_copy(ent_hbm.at[pl.ds(0, _CH)], oh, sem_h).wait()
        pltpu.make_async_copy(ent_hbm.at[pl.ds(0, _CH)], ot, sem_t).wait()

        pltpu.sync_copy(oh, h_out.at[pl.ds(base + cbase, _CH)])
        pltpu.sync_copy(ot, t_out.at[pl.ds(base + cbase, _CH)])
        pltpu.sync_copy(orel, r_out.at[pl.ds(base + cbase, _CH)])


def kernel(ent_emb, rel_emb, triples):
    ent_rm = jnp.take(ent_emb, jnp.arange(_NE, dtype=jnp.int32), axis=0)
    rel_lin = rel_emb.reshape(_NR * _D)
    h = triples[:, 0].astype(jnp.int32)
    r = triples[:, 1].astype(jnp.int32)
    t = triples[:, 2].astype(jnp.int32)
    ho, ro, to = _gather3(ent_rm, rel_lin, h, r, t)
    return ho[:, None, :], ro[:, None, :], to[:, None, :]


# speedup vs baseline: 2.7228x; 2.7228x over previous
import functools

import jax
import jax.numpy as jnp
from jax import lax
from jax.experimental import pallas as pl
from jax.experimental.pallas import tpu as pltpu
from jax.experimental.pallas import tpu_sc as plsc

_B = 16384
_D = 64
_NE = 1000000
_NR = 1000

_info = plsc.get_sparse_core_info()
_NC = _info.num_cores
_NS = _info.num_subcores
_NW = _NC * _NS
_BPW = _B // _NW
_CH = 128
_NCHUNK = _BPW // _CH

_mesh = plsc.VectorSubcoreMesh(core_axis_name="c", subcore_axis_name="s")


@functools.partial(
    pl.kernel,
    mesh=_mesh,
    out_type=[
        jax.ShapeDtypeStruct((_B, _D), jnp.float32),
        jax.ShapeDtypeStruct((_B, _D), jnp.float32),
        jax.ShapeDtypeStruct((_B, _D), jnp.float32),
    ],
    scratch_types=[
        pltpu.VMEM((_BPW,), jnp.int32),
        pltpu.VMEM((_BPW,), jnp.int32),
        pltpu.VMEM((_BPW,), jnp.int32),
        pltpu.VMEM((_CH, _D), jnp.float32),
        pltpu.VMEM((_CH, _D), jnp.float32),
        pltpu.VMEM((_CH, _D), jnp.float32),
        pltpu.VMEM((_NR * _D,), jnp.float32),
        pltpu.SemaphoreType.DMA,
        pltpu.SemaphoreType.DMA,
    ],
)
def _gather3(ent_hbm, rel_hbm, h_hbm, r_hbm, t_hbm,
             h_out, r_out, t_out,
             sh_s, st_s, sr_s,
             oh, ot, orel, rel_v,
             sem_h, sem_t):
    wid = lax.axis_index("s") * _NC + lax.axis_index("c")
    base = wid * _BPW

    pltpu.sync_copy(h_hbm.at[pl.ds(base, _BPW)], sh_s)
    pltpu.sync_copy(t_hbm.at[pl.ds(base, _BPW)], st_s)
    pltpu.sync_copy(r_hbm.at[pl.ds(base, _BPW)], sr_s)
    pltpu.sync_copy(rel_hbm, rel_v)

    @pl.loop(0, _NCHUNK)
    def _chunk(ci):
        cbase = ci * _CH

        @pl.loop(0, _CH // 16)
        def _fire(g):
            s = pl.ds(cbase + g * 16, 16)
            hv = sh_s[s]
            tv = st_s[s]
            rv = sr_s[s] * _D
            for l in range(16):
                j = g * 16 + l
                pltpu.async_copy(ent_hbm.at[hv[l]], oh.at[j], sem_h)
                pltpu.async_copy(ent_hbm.at[tv[l]], ot.at[j], sem_t)
                rbase = rv[l]
                for b in range(_D // 16):
                    orel[j, pl.ds(b * 16, 16)] = rel_v[pl.ds(rbase + b * 16, 16)]

        pltpu.make_async_copy(ent_hbm.at[pl.ds(0, _CH)], oh, sem_h).wait()
        pltpu.make_async_copy(ent_hbm.at[pl.ds(0, _CH)], ot, sem_t).wait()

        pltpu.sync_copy(oh, h_out.at[pl.ds(base + cbase, _CH)])
        pltpu.sync_copy(ot, t_out.at[pl.ds(base + cbase, _CH)])
        pltpu.sync_copy(orel, r_out.at[pl.ds(base + cbase, _CH)])


def kernel(ent_emb, rel_emb, triples):
    rel_lin = rel_emb.reshape(_NR * _D)
    h = triples[:, 0].astype(jnp.int32)
    r = triples[:, 1].astype(jnp.int32)
    t = triples[:, 2].astype(jnp.int32)
    ho, ro, to = _gather3(ent_emb, rel_lin, h, r, t)
    return ho[:, None, :], ro[:, None, :], to[:, None, :]
